# baseline (speedup 1.0000x reference)
import functools

import jax
import jax.numpy as jnp
from jax import lax
from jax.experimental import pallas as pl
from jax.experimental.pallas import tpu as pltpu
from jax.experimental.pallas import tpu_sc as plsc

_N = 100000
_H = 32
_B = 1024
_SS = 20

_NC = 2
_NS = 16
_NW = _NC * _NS
_BT = 2 * _B + 2 * _B * _SS
_BPW = _BT // _NW
_CH = 112
_NCH = _BPW // _CH
_DW = 8

_mesh = plsc.VectorSubcoreMesh(core_axis_name="c", subcore_axis_name="s")


def _dots_body(w_ref, embt_ref, s2_ref):
    W = w_ref[...]
    wsym = 0.5 * (W[:, :_H] + W[:, _H:])
    s2_ref[...] = lax.dot_general(
        wsym, embt_ref[...], (((1,), (0,)), ((), ())),
        preferred_element_type=jnp.float32,
    )


_dots_tc = pl.pallas_call(
    _dots_body,
    out_shape=jax.ShapeDtypeStruct((2, _N), jnp.float32),
    in_specs=[
        pl.BlockSpec(memory_space=pltpu.VMEM),
        pl.BlockSpec(memory_space=pltpu.VMEM),
    ],
)


@functools.partial(
    pl.kernel,
    mesh=_mesh,
    out_type=jax.ShapeDtypeStruct((_BT, _DW), jnp.float32),
    scratch_types=[
        pltpu.VMEM((_NCH, _CH), jnp.int32),
        pltpu.VMEM((_BPW, _DW), jnp.float32),
        pltpu.SemaphoreType.DMA,
    ],
    compiler_params=pltpu.CompilerParams(use_tc_tiling_on_sc=False),
)
def _gather_sc(table_hbm, idx_hbm, out_hbm, idx_v, rows_v, sem):
    wid = lax.axis_index("s") * _NC + lax.axis_index("c")
    pltpu.sync_copy(idx_hbm.at[wid], idx_v)
    copies = []
    for j in range(_NCH):
        copies.append(
            pltpu.async_copy(
                table_hbm.at[idx_v.at[j]],
                rows_v.at[pl.ds(j * _CH, _CH)],
                sem,
            )
        )
    for c in copies:
        c.wait()
    pltpu.sync_copy(rows_v, out_hbm.at[pl.ds(wid * _BPW, _BPW)])


def _softplus(g, p):
    r = jnp.clip(g / p, -75.0, 75.0)
    return p * jnp.log1p(jnp.exp(r))


def _final_body(b_ref, psi_ref, k_ref, dg_ref, lam_ref, ls_ref):
    row = lax.broadcasted_iota(jnp.int32, (32, 128), 0)
    lane = lax.broadcasted_iota(jnp.int32, (32, 128), 1)
    wsel = jnp.where(lane == 8 * (row % 16) + row // 16, 1.0, 0.0)
    S32 = lax.dot_general(
        wsel, dg_ref[...], (((1,), (1,)), ((), ())),
        preferred_element_type=jnp.float32,
    )

    b0 = b_ref[0]
    b1 = b_ref[1]
    p0 = psi_ref[0]
    p1 = psi_ref[1]

    def blk(kk, piece):
        c = piece * (_B // 16)
        return lax.slice(S32, (16 * kk, c), (16 * kk + 16, c + _B // 16))

    p0u, p1u = blk(0, 0), blk(1, 0)
    p0v, p1v = blk(0, 1), blk(1, 1)

    kb = k_ref[...]
    lam0 = _softplus(p0u + p0v + b0, p0)
    lam1 = _softplus(p1u + p1v + b1, p1)
    lam_ref[...] = jnp.where(kb == 0, lam0, lam1)

    for s in range(_SS):
        acc = (
            _softplus(p0u + blk(0, 2 + s) + b0, p0)
            + _softplus(p1u + blk(1, 2 + s) + b1, p1)
            + _softplus(p0v + blk(0, 2 + _SS + s) + b0, p0)
            + _softplus(p1v + blk(1, 2 + _SS + s) + b1, p1)
        )
        ls_ref[0, s] = jnp.sum(acc) * (1.0 / _SS)


_final_tc = pl.pallas_call(
    _final_body,
    out_shape=(
        jax.ShapeDtypeStruct((16, _B // 16), jnp.float32),
        jax.ShapeDtypeStruct((1, _SS), jnp.float32),
    ),
    in_specs=[
        pl.BlockSpec(memory_space=pltpu.SMEM),
        pl.BlockSpec(memory_space=pltpu.SMEM),
        pl.BlockSpec(memory_space=pltpu.VMEM),
        pl.BlockSpec(memory_space=pltpu.VMEM),
    ],
    out_specs=(
        pl.BlockSpec(memory_space=pltpu.VMEM),
        pl.BlockSpec(memory_space=pltpu.SMEM),
    ),
)


def kernel(embeddings, W_omega, b_omega, psi, t, u, v, k, u_others, v_others):
    del t
    s2 = _dots_tc(W_omega, embeddings.T)
    d8 = jnp.pad(s2.T, ((0, 0), (0, _DW - 2)))

    idx = jnp.concatenate([
        u.astype(jnp.int32),
        v.astype(jnp.int32),
        v_others.astype(jnp.int32).T.reshape(-1),
        u_others.astype(jnp.int32).T.reshape(-1),
    ])
    idx3 = idx.reshape(_NW, _NCH, _CH)
    dz = _gather_sc(d8, idx3)
    dg = dz.reshape(_BT // 16, 16 * _DW)

    kb = k.astype(jnp.int32).reshape(_B // 16, 16).T
    lam_blk, ls = _final_tc(b_omega, psi, kb, dg)
    lam = lam_blk.T.reshape(_B)
    return (lam, ls.reshape(_SS))


# speedup vs baseline: 3.1235x; 3.1235x over previous
import functools

import jax
import jax.numpy as jnp
from jax import lax
from jax.experimental import pallas as pl
from jax.experimental.pallas import tpu as pltpu
from jax.experimental.pallas import tpu_sc as plsc

_N = 100000
_H = 32
_B = 1024
_SS = 20

_NC = 2
_NS = 16
_NW = _NC * _NS
_BT = 2 * _B + 2 * _B * _SS
_SPP = _BT // (_NW // 2)
_L = 16

_mesh = plsc.VectorSubcoreMesh(core_axis_name="c", subcore_axis_name="s")


def _dots_body(w_ref, embt_ref, s0_ref, s1_ref):
    W = w_ref[...]
    wsym = 0.5 * (W[:, :_H] + W[:, _H:])
    S2 = lax.dot_general(
        wsym, embt_ref[...], (((1,), (0,)), ((), ())),
        preferred_element_type=jnp.float32,
    )
    s0_ref[...] = S2[0]
    s1_ref[...] = S2[1]


_dots_tc = pl.pallas_call(
    _dots_body,
    out_shape=(
        jax.ShapeDtypeStruct((_N,), jnp.float32),
        jax.ShapeDtypeStruct((_N,), jnp.float32),
    ),
    in_specs=[
        pl.BlockSpec(memory_space=pltpu.VMEM),
        pl.BlockSpec(memory_space=pltpu.VMEM),
    ],
)


@functools.partial(
    pl.kernel,
    mesh=_mesh,
    out_type=(
        jax.ShapeDtypeStruct((_BT,), jnp.float32),
        jax.ShapeDtypeStruct((_BT,), jnp.float32),
    ),
    scratch_types=[
        pltpu.VMEM((_N,), jnp.float32),
        pltpu.VMEM((_SPP,), jnp.int32),
        pltpu.VMEM((_SPP,), jnp.float32),
    ],
    compiler_params=pltpu.CompilerParams(
        use_tc_tiling_on_sc=False, needs_layout_passes=False
    ),
)
def _gather_sc(s0_hbm, s1_hbm, idx_hbm, g0_hbm, g1_hbm, s_v, idx_v, out_v):
    wid = lax.axis_index("s") * _NC + lax.axis_index("c")
    kk = wid % 2
    base = (wid // 2) * _SPP
    pltpu.sync_copy(idx_hbm.at[pl.ds(base, _SPP)], idx_v)

    @pl.when(kk == 0)
    def _():
        pltpu.sync_copy(s0_hbm, s_v)

    @pl.when(kk == 1)
    def _():
        pltpu.sync_copy(s1_hbm, s_v)

    for c in range(_SPP // _L):
        iv = idx_v[pl.ds(c * _L, _L)]
        out_v[pl.ds(c * _L, _L)] = plsc.load_gather(s_v, [iv])

    @pl.when(kk == 0)
    def _():
        pltpu.sync_copy(out_v, g0_hbm.at[pl.ds(base, _SPP)])

    @pl.when(kk == 1)
    def _():
        pltpu.sync_copy(out_v, g1_hbm.at[pl.ds(base, _SPP)])


def _softplus(g, p):
    r = jnp.clip(g / p, -75.0, 75.0)
    return p * jnp.log1p(jnp.exp(r))


def _final_body(b_ref, psi_ref, k_ref, g0_ref, g1_ref, lam_ref, ls_ref):
    s0 = g0_ref[...]
    s1 = g1_ref[...]
    b0 = b_ref[0]
    b1 = b_ref[1]
    p0 = psi_ref[0]
    p1 = psi_ref[1]

    su0 = lax.slice(s0, (0,), (_B,))
    su1 = lax.slice(s1, (0,), (_B,))
    sv0 = lax.slice(s0, (_B,), (2 * _B,))
    sv1 = lax.slice(s1, (_B,), (2 * _B,))

    kk = k_ref[...]
    lam0 = _softplus(su0 + sv0 + b0, p0)
    lam1 = _softplus(su1 + sv1 + b1, p1)
    lam_ref[...] = jnp.where(kk == 0, lam0, lam1)

    ovo = 2 * _B
    ouo = ovo + _SS * _B
    for s in range(_SS):
        cv = ovo + s * _B
        cu = ouo + s * _B
        acc = (
            _softplus(su0 + lax.slice(s0, (cv,), (cv + _B,)) + b0, p0)
            + _softplus(su1 + lax.slice(s1, (cv,), (cv + _B,)) + b1, p1)
            + _softplus(sv0 + lax.slice(s0, (cu,), (cu + _B,)) + b0, p0)
            + _softplus(sv1 + lax.slice(s1, (cu,), (cu + _B,)) + b1, p1)
        )
        ls_ref[0, s] = jnp.sum(acc) * (1.0 / _SS)


_final_tc = pl.pallas_call(
    _final_body,
    out_shape=(
        jax.ShapeDtypeStruct((_B,), jnp.float32),
        jax.ShapeDtypeStruct((1, _SS), jnp.float32),
    ),
    in_specs=[
        pl.BlockSpec(memory_space=pltpu.SMEM),
        pl.BlockSpec(memory_space=pltpu.SMEM),
        pl.BlockSpec(memory_space=pltpu.VMEM),
        pl.BlockSpec(memory_space=pltpu.VMEM),
        pl.BlockSpec(memory_space=pltpu.VMEM),
    ],
    out_specs=(
        pl.BlockSpec(memory_space=pltpu.VMEM),
        pl.BlockSpec(memory_space=pltpu.SMEM),
    ),
)


def kernel(embeddings, W_omega, b_omega, psi, t, u, v, k, u_others, v_others):
    del t
    s0, s1 = _dots_tc(W_omega, embeddings.T)

    idx = jnp.concatenate([
        u.astype(jnp.int32),
        v.astype(jnp.int32),
        v_others.astype(jnp.int32).T.reshape(-1),
        u_others.astype(jnp.int32).T.reshape(-1),
    ])
    g0, g1 = _gather_sc(s0, s1, idx)
    lam, ls = _final_tc(b_omega, psi, k.astype(jnp.int32), g0, g1)
    return (lam, ls.reshape(_SS))


# speedup vs baseline: 3.2116x; 1.0282x over previous
import functools

import jax
import jax.numpy as jnp
from jax import lax
from jax.experimental import pallas as pl
from jax.experimental.pallas import tpu as pltpu
from jax.experimental.pallas import tpu_sc as plsc

_N = 100000
_H = 32
_B = 1024
_SS = 20

_NC = 2
_NS = 16
_NW = _NC * _NS
_BT = 2 * _B + 2 * _B * _SS
_SPP = _BT // (_NW // 2)
_L = 16

_mesh = plsc.VectorSubcoreMesh(core_axis_name="c", subcore_axis_name="s")


def _dots_body(w_ref, embt_ref, s0_ref, s1_ref):
    W = w_ref[...]
    wsym = 0.5 * (W[:, :_H] + W[:, _H:])
    S2 = lax.dot_general(
        wsym, embt_ref[...], (((1,), (0,)), ((), ())),
        preferred_element_type=jnp.float32,
    )
    s0_ref[...] = S2[0]
    s1_ref[...] = S2[1]


_dots_tc = pl.pallas_call(
    _dots_body,
    out_shape=(
        jax.ShapeDtypeStruct((_N,), jnp.float32),
        jax.ShapeDtypeStruct((_N,), jnp.float32),
    ),
    in_specs=[
        pl.BlockSpec(memory_space=pltpu.VMEM),
        pl.BlockSpec(memory_space=pltpu.VMEM),
    ],
)


_NH = _N // 2
_SPT = _BT // 8


@functools.partial(
    pl.kernel,
    mesh=_mesh,
    out_type=(
        jax.ShapeDtypeStruct((_BT,), jnp.float32),
        jax.ShapeDtypeStruct((_BT,), jnp.float32),
        jax.ShapeDtypeStruct((_BT,), jnp.float32),
        jax.ShapeDtypeStruct((_BT,), jnp.float32),
    ),
    scratch_types=[
        pltpu.VMEM((_NH,), jnp.float32),
        pltpu.VMEM((_SPT,), jnp.int32),
        pltpu.VMEM((_SPT,), jnp.float32),
    ],
    compiler_params=pltpu.CompilerParams(
        use_tc_tiling_on_sc=False, needs_layout_passes=False
    ),
)
def _gather_sc(s0_hbm, s1_hbm, idx_hbm, g0a_hbm, g0b_hbm, g1a_hbm, g1b_hbm,
               s_v, idx_v, out_v):
    wid = lax.axis_index("s") * _NC + lax.axis_index("c")
    kk = wid % 2
    half = (wid // 2) % 2
    base = (wid // 4) * _SPT
    lo = half * _NH
    pltpu.sync_copy(idx_hbm.at[pl.ds(base, _SPT)], idx_v)

    @pl.when(kk == 0)
    def _():
        pltpu.sync_copy(s0_hbm.at[pl.ds(lo, _NH)], s_v)

    @pl.when(kk == 1)
    def _():
        pltpu.sync_copy(s1_hbm.at[pl.ds(lo, _NH)], s_v)

    zeros = jnp.zeros((_L,), jnp.float32)
    for c in range(_SPT // _L):
        iv = idx_v[pl.ds(c * _L, _L)] - lo
        mask = (iv >= 0) & (iv < _NH)
        ivc = jnp.clip(iv, 0, _NH - 1)
        vals = jnp.where(mask, plsc.load_gather(s_v, [ivc]), zeros)
        out_v[pl.ds(c * _L, _L)] = vals

    out = [[g0a_hbm, g0b_hbm], [g1a_hbm, g1b_hbm]]
    for kx in (0, 1):
        for hx in (0, 1):
            @pl.when((kk == kx) & (half == hx))
            def _(o=out[kx][hx]):
                pltpu.sync_copy(out_v, o.at[pl.ds(base, _SPT)])


def _softplus(g, p):
    r = jnp.clip(g / p, -75.0, 75.0)
    return p * jnp.log1p(jnp.exp(r))


def _final_body(b_ref, psi_ref, k_ref, g0a_ref, g0b_ref, g1a_ref, g1b_ref,
                lam_ref, ls_ref):
    s0 = g0a_ref[...] + g0b_ref[...]
    s1 = g1a_ref[...] + g1b_ref[...]
    b0 = b_ref[0]
    b1 = b_ref[1]
    p0 = psi_ref[0]
    p1 = psi_ref[1]

    su0 = lax.slice(s0, (0,), (_B,))
    su1 = lax.slice(s1, (0,), (_B,))
    sv0 = lax.slice(s0, (_B,), (2 * _B,))
    sv1 = lax.slice(s1, (_B,), (2 * _B,))

    kk = k_ref[...]
    lam0 = _softplus(su0 + sv0 + b0, p0)
    lam1 = _softplus(su1 + sv1 + b1, p1)
    lam_ref[...] = jnp.where(kk == 0, lam0, lam1)

    ovo = 2 * _B
    ouo = ovo + _SS * _B
    for s in range(_SS):
        cv = ovo + s * _B
        cu = ouo + s * _B
        acc = (
            _softplus(su0 + lax.slice(s0, (cv,), (cv + _B,)) + b0, p0)
            + _softplus(su1 + lax.slice(s1, (cv,), (cv + _B,)) + b1, p1)
            + _softplus(sv0 + lax.slice(s0, (cu,), (cu + _B,)) + b0, p0)
            + _softplus(sv1 + lax.slice(s1, (cu,), (cu + _B,)) + b1, p1)
        )
        ls_ref[0, s] = jnp.sum(acc) * (1.0 / _SS)


_final_tc = pl.pallas_call(
    _final_body,
    out_shape=(
        jax.ShapeDtypeStruct((_B,), jnp.float32),
        jax.ShapeDtypeStruct((1, _SS), jnp.float32),
    ),
    in_specs=[
        pl.BlockSpec(memory_space=pltpu.SMEM),
        pl.BlockSpec(memory_space=pltpu.SMEM),
        pl.BlockSpec(memory_space=pltpu.VMEM),
        pl.BlockSpec(memory_space=pltpu.VMEM),
        pl.BlockSpec(memory_space=pltpu.VMEM),
        pl.BlockSpec(memory_space=pltpu.VMEM),
        pl.BlockSpec(memory_space=pltpu.VMEM),
    ],
    out_specs=(
        pl.BlockSpec(memory_space=pltpu.VMEM),
        pl.BlockSpec(memory_space=pltpu.SMEM),
    ),
)


def kernel(embeddings, W_omega, b_omega, psi, t, u, v, k, u_others, v_others):
    del t
    s0, s1 = _dots_tc(W_omega, embeddings.T)

    idx = jnp.concatenate([
        u.astype(jnp.int32),
        v.astype(jnp.int32),
        v_others.astype(jnp.int32).T.reshape(-1),
        u_others.astype(jnp.int32).T.reshape(-1),
    ])
    g0a, g0b, g1a, g1b = _gather_sc(s0, s1, idx)
    lam, ls = _final_tc(b_omega, psi, k.astype(jnp.int32), g0a, g0b, g1a, g1b)
    return (lam, ls.reshape(_SS))
